# baseline (speedup 1.0000x reference)
import functools

import jax
import jax.numpy as jnp
from jax import lax
from jax.experimental import pallas as pl
from jax.experimental.pallas import tpu as pltpu
from jax.experimental.pallas import tpu_sc as plsc

N_NODES = 10000
N_EDGES = 320000
NUM_CORES = 2
NUM_SUBCORES = 16
EDGES_PER_SUBCORE = N_EDGES // NUM_SUBCORES
CHUNK = 125
N_CHUNKS = EDGES_PER_SUBCORE // CHUNK
ROWS_PER_SUBCORE = N_NODES // NUM_SUBCORES
ZERO_ROWS = 25


def _sc_aggregate(h_split, src, dst, fh):
    mesh = plsc.VectorSubcoreMesh(core_axis_name="c", subcore_axis_name="s")

    @functools.partial(
        pl.kernel,
        out_type=jax.ShapeDtypeStruct((N_NODES, NUM_CORES * fh), jnp.float32),
        mesh=mesh,
        compiler_params=pltpu.CompilerParams(use_tc_tiling_on_sc=False),
        scratch_types=[
            pltpu.VMEM((N_CHUNKS, CHUNK), jnp.int32),
            pltpu.VMEM((N_CHUNKS, CHUNK), jnp.int32),
            pltpu.VMEM((CHUNK, fh), jnp.float32),
            pltpu.VMEM((CHUNK, fh), jnp.float32),
            pltpu.VMEM((ZERO_ROWS, fh), jnp.float32),
            pltpu.VMEM_SHARED((N_NODES, fh), jnp.float32),
            pltpu.SemaphoreType.DMA,
            pltpu.SemaphoreType.DMA,
        ],
    )
    def agg(h_hbm, src_hbm, dst_hbm, out_hbm,
            src_v, dst_v, rows0, rows1, zb_v, acc_sh, gs0, gs1):
        c = lax.axis_index("c")
        s = lax.axis_index("s")

        @pl.loop(0, ZERO_ROWS)
        def _(i):
            @pl.loop(0, fh, step=16)
            def _(j):
                zb_v[pl.ds(i, 1), pl.ds(j, 16)] = jnp.zeros((1, 16), jnp.float32)

        row0 = s * ROWS_PER_SUBCORE

        @pl.loop(0, ROWS_PER_SUBCORE, step=ZERO_ROWS)
        def _(r):
            pltpu.sync_copy(zb_v, acc_sh.at[pl.ds(row0 + r, ZERO_ROWS)])

        pltpu.sync_copy(src_hbm.at[s], src_v)
        pltpu.sync_copy(dst_hbm.at[s], dst_v)

        plsc.subcore_barrier()

        hc = h_hbm.at[c]
        pltpu.async_copy(hc.at[src_v.at[0]], rows0, gs0)

        @pl.loop(0, N_CHUNKS, step=2)
        def _(i):
            pltpu.async_copy(hc.at[src_v.at[i + 1]], rows1, gs1)
            pltpu.make_async_copy(hc.at[src_v.at[i]], rows0, gs0).wait()
            pltpu.sync_copy(rows0, acc_sh.at[dst_v.at[i]], add=True)

            @pl.when(i + 2 < N_CHUNKS)
            def _():
                pltpu.async_copy(hc.at[src_v.at[i + 2]], rows0, gs0)
            pltpu.make_async_copy(hc.at[src_v.at[i + 1]], rows1, gs1).wait()
            pltpu.sync_copy(rows1, acc_sh.at[dst_v.at[i + 1]], add=True)

        plsc.subcore_barrier()

        out0 = s * 632
        col0 = c * fh

        @pl.when(s < NUM_SUBCORES - 1)
        def _():
            pltpu.sync_copy(acc_sh.at[pl.ds(out0, 632)],
                            out_hbm.at[pl.ds(out0, 632), pl.ds(col0, fh)])

        @pl.when(s == NUM_SUBCORES - 1)
        def _():
            pltpu.sync_copy(acc_sh.at[pl.ds(out0, 520)],
                            out_hbm.at[pl.ds(out0, 520), pl.ds(col0, fh)])

    return agg(h_split, src, dst)


def _tc_matmul_split(x, W, relu_x=False):
    m, k = x.shape
    n = W.shape[1]
    n2 = n // NUM_CORES
    bm = 1000
    w_split = jnp.stack([W[:, :n2], W[:, n2:]])

    def body(x_ref, w_ref, o_ref):
        t = x_ref[...]
        if relu_x:
            t = jnp.maximum(t, 0.0)
        o_ref[0] = jnp.dot(t, w_ref[0], preferred_element_type=jnp.float32)

    return pl.pallas_call(
        body,
        grid=(m // bm, NUM_CORES),
        in_specs=[
            pl.BlockSpec((bm, k), lambda i, j: (i, 0)),
            pl.BlockSpec((1, k, n2), lambda i, j: (j, 0, 0)),
        ],
        out_specs=pl.BlockSpec((1, bm, n2), lambda i, j: (j, i, 0)),
        out_shape=jax.ShapeDtypeStruct((NUM_CORES, m, n2), jnp.float32),
    )(x, w_split)


def kernel(x, edge_index, W1, W2):
    ei = edge_index.astype(jnp.int32)
    src = ei[0].reshape(NUM_SUBCORES, N_CHUNKS, CHUNK)
    dst = ei[1].reshape(NUM_SUBCORES, N_CHUNKS, CHUNK)
    h1 = _tc_matmul_split(x, W1)
    p = _sc_aggregate(h1, src, dst, 64)
    h2 = _tc_matmul_split(p, W2, relu_x=True)
    return _sc_aggregate(h2, src, dst, 32)


# speedup vs baseline: 1.2472x; 1.2472x over previous
import functools

import jax
import jax.numpy as jnp
from jax import lax
from jax.experimental import pallas as pl
from jax.experimental.pallas import tpu as pltpu
from jax.experimental.pallas import tpu_sc as plsc

N_NODES = 10000
N_EDGES = 320000
NUM_CORES = 2
NUM_SUBCORES = 16
NUM_TILES = NUM_CORES * NUM_SUBCORES
EDGES_PER_TILE = N_EDGES // NUM_TILES
CHUNK = 100
N_CHUNKS = EDGES_PER_TILE // CHUNK
N_HALF = N_CHUNKS // 2
ROWS_PER_SUBCORE = N_NODES // NUM_SUBCORES


def _sc_aggregate(h, src, dst, feat):
    mesh = plsc.VectorSubcoreMesh(core_axis_name="c", subcore_axis_name="s")

    @functools.partial(
        pl.kernel,
        out_type=jax.ShapeDtypeStruct((NUM_CORES, N_NODES, feat), jnp.float32),
        mesh=mesh,
        compiler_params=pltpu.CompilerParams(use_tc_tiling_on_sc=False),
        scratch_types=[
            pltpu.VMEM((N_HALF, CHUNK), jnp.int32),
            pltpu.VMEM((N_HALF, CHUNK), jnp.int32),
            pltpu.VMEM((CHUNK, feat), jnp.float32),
            pltpu.VMEM((CHUNK, feat), jnp.float32),
            pltpu.VMEM((CHUNK, feat), jnp.float32),
            pltpu.VMEM_SHARED((N_NODES, feat), jnp.float32),
            pltpu.SemaphoreType.DMA,
            pltpu.SemaphoreType.DMA,
            pltpu.SemaphoreType.DMA,
        ],
    )
    def agg(h_hbm, src_hbm, dst_hbm, out_hbm,
            src_v, dst_v, rows0, rows1, rows2, acc_sh, gs0, gs1, gs2):
        c = lax.axis_index("c")
        s = lax.axis_index("s")
        wid = s * NUM_CORES + c

        @pl.loop(0, CHUNK)
        def _(i):
            @pl.loop(0, feat, step=16)
            def _(j):
                rows0[pl.ds(i, 1), pl.ds(j, 16)] = jnp.zeros((1, 16), jnp.float32)

        row0 = s * ROWS_PER_SUBCORE

        @pl.loop(0, ROWS_PER_SUBCORE - CHUNK, step=CHUNK)
        def _(r):
            pltpu.sync_copy(rows0, acc_sh.at[pl.ds(row0 + r, CHUNK)])
        rem = ROWS_PER_SUBCORE % CHUNK
        pltpu.sync_copy(rows0.at[pl.ds(0, rem)],
                        acc_sh.at[pl.ds(row0 + ROWS_PER_SUBCORE - rem, rem)])

        plsc.subcore_barrier()

        for half in range(2):
            pltpu.sync_copy(src_hbm.at[wid, pl.ds(half * N_HALF, N_HALF)], src_v)
            pltpu.sync_copy(dst_hbm.at[wid, pl.ds(half * N_HALF, N_HALF)], dst_v)

            pltpu.async_copy(h_hbm.at[src_v.at[0]], rows0, gs0)
            pltpu.async_copy(h_hbm.at[src_v.at[1]], rows1, gs1)

            @pl.loop(0, N_HALF - 2, step=3)
            def _(i):
                pltpu.async_copy(h_hbm.at[src_v.at[i + 2]], rows2, gs2)
                pltpu.make_async_copy(h_hbm.at[src_v.at[i]], rows0, gs0).wait()
                pltpu.sync_copy(rows0, acc_sh.at[dst_v.at[i]], add=True)

                pltpu.async_copy(h_hbm.at[src_v.at[i + 3]], rows0, gs0)
                pltpu.make_async_copy(h_hbm.at[src_v.at[i + 1]], rows1, gs1).wait()
                pltpu.sync_copy(rows1, acc_sh.at[dst_v.at[i + 1]], add=True)

                pltpu.async_copy(h_hbm.at[src_v.at[i + 4]], rows1, gs1)
                pltpu.make_async_copy(h_hbm.at[src_v.at[i + 2]], rows2, gs2).wait()
                pltpu.sync_copy(rows2, acc_sh.at[dst_v.at[i + 2]], add=True)

            pltpu.make_async_copy(h_hbm.at[src_v.at[N_HALF - 2]], rows0, gs0).wait()
            pltpu.sync_copy(rows0, acc_sh.at[dst_v.at[N_HALF - 2]], add=True)
            pltpu.make_async_copy(h_hbm.at[src_v.at[N_HALF - 1]], rows1, gs1).wait()
            pltpu.sync_copy(rows1, acc_sh.at[dst_v.at[N_HALF - 1]], add=True)

        plsc.subcore_barrier()

        out0 = s * 632

        @pl.when(s < NUM_SUBCORES - 1)
        def _():
            pltpu.sync_copy(acc_sh.at[pl.ds(out0, 632)],
                            out_hbm.at[c, pl.ds(out0, 632)])

        @pl.when(s == NUM_SUBCORES - 1)
        def _():
            pltpu.sync_copy(acc_sh.at[pl.ds(out0, 520)],
                            out_hbm.at[c, pl.ds(out0, 520)])

    return agg(h, src, dst)


def _tc_matmul(x, W):
    m, k = x.shape
    n = W.shape[1]
    bm = 1000

    def body(x_ref, w_ref, o_ref):
        o_ref[...] = jnp.dot(x_ref[...], w_ref[...],
                             preferred_element_type=jnp.float32)

    return pl.pallas_call(
        body,
        grid=(m // bm,),
        in_specs=[
            pl.BlockSpec((bm, k), lambda i: (i, 0)),
            pl.BlockSpec((k, n), lambda i: (0, 0)),
        ],
        out_specs=pl.BlockSpec((bm, n), lambda i: (i, 0)),
        out_shape=jax.ShapeDtypeStruct((m, n), jnp.float32),
    )(x, W)


def _tc_relu_add_matmul(p0, p1, W):
    m, k = p0.shape
    n = W.shape[1]
    bm = 1000

    def body(a_ref, b_ref, w_ref, o_ref):
        t = jnp.maximum(a_ref[...] + b_ref[...], 0.0)
        o_ref[...] = jnp.dot(t, w_ref[...], preferred_element_type=jnp.float32)

    return pl.pallas_call(
        body,
        grid=(m // bm,),
        in_specs=[
            pl.BlockSpec((bm, k), lambda i: (i, 0)),
            pl.BlockSpec((bm, k), lambda i: (i, 0)),
            pl.BlockSpec((k, n), lambda i: (0, 0)),
        ],
        out_specs=pl.BlockSpec((bm, n), lambda i: (i, 0)),
        out_shape=jax.ShapeDtypeStruct((m, n), jnp.float32),
    )(p0, p1, W)


def _tc_add(a, b):
    m, n = a.shape

    def body(a_ref, b_ref, o_ref):
        o_ref[...] = a_ref[...] + b_ref[...]

    return pl.pallas_call(
        body,
        out_shape=jax.ShapeDtypeStruct((m, n), jnp.float32),
    )(a, b)


def kernel(x, edge_index, W1, W2):
    ei = edge_index.astype(jnp.int32)
    src = ei[0].reshape(NUM_TILES, N_CHUNKS, CHUNK)
    dst = ei[1].reshape(NUM_TILES, N_CHUNKS, CHUNK)
    h1 = _tc_matmul(x, W1)
    p = _sc_aggregate(h1, src, dst, 128)
    h2 = _tc_relu_add_matmul(p[0], p[1], W2)
    q = _sc_aggregate(h2, src, dst, 64)
    return _tc_add(q[0], q[1])


# speedup vs baseline: 1.2707x; 1.0188x over previous
import functools

import jax
import jax.numpy as jnp
from jax import lax
from jax.experimental import pallas as pl
from jax.experimental.pallas import tpu as pltpu
from jax.experimental.pallas import tpu_sc as plsc

N_NODES = 10000
N_EDGES = 320000
NUM_CORES = 2
NUM_SUBCORES = 16
NUM_TILES = NUM_CORES * NUM_SUBCORES
EDGES_PER_TILE = N_EDGES // NUM_TILES
CHUNK = 100
N_CHUNKS = EDGES_PER_TILE // CHUNK
N_HALF = N_CHUNKS // 2
ROWS_PER_SUBCORE = N_NODES // NUM_SUBCORES


def _sc_aggregate(h, src, dst, feat):
    mesh = plsc.VectorSubcoreMesh(core_axis_name="c", subcore_axis_name="s")
    nb = 3 if feat == 128 else 5

    @functools.partial(
        pl.kernel,
        out_type=jax.ShapeDtypeStruct((NUM_CORES, N_NODES, feat), jnp.float32),
        mesh=mesh,
        compiler_params=pltpu.CompilerParams(use_tc_tiling_on_sc=False),
        scratch_types=(
            [pltpu.VMEM((N_HALF, CHUNK), jnp.int32),
             pltpu.VMEM((N_HALF, CHUNK), jnp.int32)]
            + [pltpu.VMEM((CHUNK, feat), jnp.float32) for _ in range(nb)]
            + [pltpu.VMEM_SHARED((N_NODES, feat), jnp.float32)]
            + [pltpu.SemaphoreType.DMA for _ in range(nb)]
        ),
    )
    def agg(h_hbm, src_hbm, dst_hbm, out_hbm, src_v, dst_v, *rest):
        rows = rest[:nb]
        acc_sh = rest[nb]
        sems = rest[nb + 1:]
        c = lax.axis_index("c")
        s = lax.axis_index("s")
        wid = s * NUM_CORES + c

        @pl.loop(0, CHUNK)
        def _(i):
            @pl.loop(0, feat, step=16)
            def _(j):
                rows[0][pl.ds(i, 1), pl.ds(j, 16)] = jnp.zeros((1, 16),
                                                               jnp.float32)

        row0 = s * ROWS_PER_SUBCORE

        @pl.loop(0, ROWS_PER_SUBCORE - CHUNK, step=CHUNK)
        def _(r):
            pltpu.sync_copy(rows[0], acc_sh.at[pl.ds(row0 + r, CHUNK)])
        rem = ROWS_PER_SUBCORE % CHUNK
        pltpu.sync_copy(rows[0].at[pl.ds(0, rem)],
                        acc_sh.at[pl.ds(row0 + ROWS_PER_SUBCORE - rem, rem)])

        plsc.subcore_barrier()

        for half in range(2):
            pltpu.sync_copy(src_hbm.at[wid, pl.ds(half * N_HALF, N_HALF)], src_v)
            pltpu.sync_copy(dst_hbm.at[wid, pl.ds(half * N_HALF, N_HALF)], dst_v)

            for j in range(nb):
                pltpu.async_copy(h_hbm.at[src_v.at[j]], rows[j], sems[j])

            @pl.loop(0, N_HALF, step=nb)
            def _(i):
                for j in range(nb):
                    @pl.when(i + j < N_HALF)
                    def _(j=j):
                        pltpu.make_async_copy(h_hbm.at[src_v.at[i + j]],
                                              rows[j], sems[j]).wait()
                        pltpu.sync_copy(rows[j], acc_sh.at[dst_v.at[i + j]],
                                        add=True)

                        @pl.when(i + j + nb < N_HALF)
                        def _():
                            pltpu.async_copy(h_hbm.at[src_v.at[i + j + nb]],
                                             rows[j], sems[j])

        plsc.subcore_barrier()

        out0 = s * 632

        @pl.when(s < NUM_SUBCORES - 1)
        def _():
            pltpu.sync_copy(acc_sh.at[pl.ds(out0, 632)],
                            out_hbm.at[c, pl.ds(out0, 632)])

        @pl.when(s == NUM_SUBCORES - 1)
        def _():
            pltpu.sync_copy(acc_sh.at[pl.ds(out0, 520)],
                            out_hbm.at[c, pl.ds(out0, 520)])

    return agg(h, src, dst)


def _tc_matmul(x, W):
    m, k = x.shape
    n = W.shape[1]
    bm = 1000

    def body(x_ref, w_ref, o_ref):
        o_ref[...] = jnp.dot(x_ref[...], w_ref[...],
                             preferred_element_type=jnp.float32)

    return pl.pallas_call(
        body,
        grid=(m // bm,),
        in_specs=[
            pl.BlockSpec((bm, k), lambda i: (i, 0)),
            pl.BlockSpec((k, n), lambda i: (0, 0)),
        ],
        out_specs=pl.BlockSpec((bm, n), lambda i: (i, 0)),
        out_shape=jax.ShapeDtypeStruct((m, n), jnp.float32),
    )(x, W)


def _tc_relu_add_matmul(p0, p1, W):
    m, k = p0.shape
    n = W.shape[1]
    bm = 1000

    def body(a_ref, b_ref, w_ref, o_ref):
        t = jnp.maximum(a_ref[...] + b_ref[...], 0.0)
        o_ref[...] = jnp.dot(t, w_ref[...], preferred_element_type=jnp.float32)

    return pl.pallas_call(
        body,
        grid=(m // bm,),
        in_specs=[
            pl.BlockSpec((bm, k), lambda i: (i, 0)),
            pl.BlockSpec((bm, k), lambda i: (i, 0)),
            pl.BlockSpec((k, n), lambda i: (0, 0)),
        ],
        out_specs=pl.BlockSpec((bm, n), lambda i: (i, 0)),
        out_shape=jax.ShapeDtypeStruct((m, n), jnp.float32),
    )(p0, p1, W)


def _tc_add(a, b):
    m, n = a.shape

    def body(a_ref, b_ref, o_ref):
        o_ref[...] = a_ref[...] + b_ref[...]

    return pl.pallas_call(
        body,
        out_shape=jax.ShapeDtypeStruct((m, n), jnp.float32),
    )(a, b)


def kernel(x, edge_index, W1, W2):
    ei = edge_index.astype(jnp.int32)
    src = ei[0].reshape(NUM_TILES, N_CHUNKS, CHUNK)
    dst = ei[1].reshape(NUM_TILES, N_CHUNKS, CHUNK)
    h1 = _tc_matmul(x, W1)
    p = _sc_aggregate(h1, src, dst, 128)
    h2 = _tc_relu_add_matmul(p[0], p[1], W2)
    q = _sc_aggregate(h2, src, dst, 64)
    return _tc_add(q[0], q[1])


# speedup vs baseline: 1.2882x; 1.0138x over previous
import functools

import jax
import jax.numpy as jnp
from jax import lax
from jax.experimental import pallas as pl
from jax.experimental.pallas import tpu as pltpu
from jax.experimental.pallas import tpu_sc as plsc

N_NODES = 10000
N_EDGES = 320000
NUM_CORES = 2
NUM_SUBCORES = 16
NUM_TILES = NUM_CORES * NUM_SUBCORES
EDGES_PER_TILE = N_EDGES // NUM_TILES
CHUNK = 100
N_CHUNKS = EDGES_PER_TILE // CHUNK
N_HALF = N_CHUNKS // 2
ROWS_PER_SUBCORE = N_NODES // NUM_SUBCORES


def _sc_aggregate(h, src, dst, feat):
    mesh = plsc.VectorSubcoreMesh(core_axis_name="c", subcore_axis_name="s")
    nb = 3 if feat == 128 else 6

    @functools.partial(
        pl.kernel,
        out_type=jax.ShapeDtypeStruct((NUM_CORES, N_NODES, feat), jnp.float32),
        mesh=mesh,
        compiler_params=pltpu.CompilerParams(use_tc_tiling_on_sc=False),
        scratch_types=(
            [pltpu.VMEM((N_HALF, CHUNK), jnp.int32),
             pltpu.VMEM((N_HALF, CHUNK), jnp.int32)]
            + [pltpu.VMEM((CHUNK, feat), jnp.float32) for _ in range(nb)]
            + [pltpu.VMEM_SHARED((N_NODES, feat), jnp.float32)]
            + [pltpu.SemaphoreType.DMA for _ in range(nb)]
        ),
    )
    def agg(h_hbm, src_hbm, dst_hbm, out_hbm, src_v, dst_v, *rest):
        rows = rest[:nb]
        acc_sh = rest[nb]
        sems = rest[nb + 1:]
        c = lax.axis_index("c")
        s = lax.axis_index("s")
        wid = s * NUM_CORES + c

        pltpu.async_copy(src_hbm.at[wid, pl.ds(0, N_HALF)], src_v, sems[0])
        pltpu.async_copy(dst_hbm.at[wid, pl.ds(0, N_HALF)], dst_v, sems[1])

        @pl.loop(0, CHUNK)
        def _(i):
            @pl.loop(0, feat, step=16)
            def _(j):
                rows[0][pl.ds(i, 1), pl.ds(j, 16)] = jnp.zeros((1, 16),
                                                               jnp.float32)

        row0 = s * ROWS_PER_SUBCORE

        @pl.loop(0, ROWS_PER_SUBCORE - CHUNK, step=CHUNK)
        def _(r):
            pltpu.sync_copy(rows[0], acc_sh.at[pl.ds(row0 + r, CHUNK)])
        rem = ROWS_PER_SUBCORE % CHUNK
        pltpu.sync_copy(rows[0].at[pl.ds(0, rem)],
                        acc_sh.at[pl.ds(row0 + ROWS_PER_SUBCORE - rem, rem)])

        plsc.subcore_barrier()

        for half in range(2):
            if half == 0:
                pltpu.make_async_copy(
                    src_hbm.at[wid, pl.ds(0, N_HALF)], src_v, sems[0]).wait()
                pltpu.make_async_copy(
                    dst_hbm.at[wid, pl.ds(0, N_HALF)], dst_v, sems[1]).wait()
            else:
                pltpu.sync_copy(src_hbm.at[wid, pl.ds(N_HALF, N_HALF)], src_v)
                pltpu.sync_copy(dst_hbm.at[wid, pl.ds(N_HALF, N_HALF)], dst_v)

            for j in range(nb):
                pltpu.async_copy(h_hbm.at[src_v.at[j]], rows[j], sems[j])

            @pl.loop(0, N_HALF, step=nb)
            def _(i):
                for j in range(nb):
                    @pl.when(i + j < N_HALF)
                    def _(j=j):
                        pltpu.make_async_copy(h_hbm.at[src_v.at[i + j]],
                                              rows[j], sems[j]).wait()
                        pltpu.sync_copy(rows[j], acc_sh.at[dst_v.at[i + j]],
                                        add=True)

                        @pl.when(i + j + nb < N_HALF)
                        def _():
                            pltpu.async_copy(h_hbm.at[src_v.at[i + j + nb]],
                                             rows[j], sems[j])

        plsc.subcore_barrier()

        out0 = s * 632

        @pl.when(s < NUM_SUBCORES - 1)
        def _():
            pltpu.sync_copy(acc_sh.at[pl.ds(out0, 632)],
                            out_hbm.at[c, pl.ds(out0, 632)])

        @pl.when(s == NUM_SUBCORES - 1)
        def _():
            pltpu.sync_copy(acc_sh.at[pl.ds(out0, 520)],
                            out_hbm.at[c, pl.ds(out0, 520)])

    return agg(h, src, dst)


def _tc_matmul(x, W):
    m, k = x.shape
    n = W.shape[1]
    bm = 1000

    def body(x_ref, w_ref, o_ref):
        o_ref[...] = jnp.dot(x_ref[...], w_ref[...],
                             preferred_element_type=jnp.float32)

    return pl.pallas_call(
        body,
        grid=(m // bm,),
        in_specs=[
            pl.BlockSpec((bm, k), lambda i: (i, 0)),
            pl.BlockSpec((k, n), lambda i: (0, 0)),
        ],
        out_specs=pl.BlockSpec((bm, n), lambda i: (i, 0)),
        out_shape=jax.ShapeDtypeStruct((m, n), jnp.float32),
    )(x, W)


def _tc_relu_add_matmul(p0, p1, W):
    m, k = p0.shape
    n = W.shape[1]
    bm = 1000

    def body(a_ref, b_ref, w_ref, o_ref):
        t = jnp.maximum(a_ref[...] + b_ref[...], 0.0)
        o_ref[...] = jnp.dot(t, w_ref[...], preferred_element_type=jnp.float32)

    return pl.pallas_call(
        body,
        grid=(m // bm,),
        in_specs=[
            pl.BlockSpec((bm, k), lambda i: (i, 0)),
            pl.BlockSpec((bm, k), lambda i: (i, 0)),
            pl.BlockSpec((k, n), lambda i: (0, 0)),
        ],
        out_specs=pl.BlockSpec((bm, n), lambda i: (i, 0)),
        out_shape=jax.ShapeDtypeStruct((m, n), jnp.float32),
    )(p0, p1, W)


def _tc_add(a, b):
    m, n = a.shape

    def body(a_ref, b_ref, o_ref):
        o_ref[...] = a_ref[...] + b_ref[...]

    return pl.pallas_call(
        body,
        out_shape=jax.ShapeDtypeStruct((m, n), jnp.float32),
    )(a, b)


def kernel(x, edge_index, W1, W2):
    ei = edge_index.astype(jnp.int32)
    src = ei[0].reshape(NUM_TILES, N_CHUNKS, CHUNK)
    dst = ei[1].reshape(NUM_TILES, N_CHUNKS, CHUNK)
    h1 = _tc_matmul(x, W1)
    p = _sc_aggregate(h1, src, dst, 128)
    h2 = _tc_relu_add_matmul(p[0], p[1], W2)
    q = _sc_aggregate(h2, src, dst, 64)
    return _tc_add(q[0], q[1])
